# baseline (speedup 1.0000x reference)
import jax
import jax.numpy as jnp
from jax.experimental import pallas as pl
from jax.experimental.pallas import tpu as pltpu

E = 8
D = 1024
R = 16
DQ = 1024
DV = 1024
SCALE = 32.0 / 16.0
ER = E * R


def _router_lora_kernel(h_ref, wr_ref, qa_ref, qb_ref, va_ref, vb_ref,
                        q_out_ref, v_out_ref,
                        wrt_ref, qa_flat_ref, va_flat_ref):
    @pl.when(pl.program_id(0) == 0)
    def _build_weight_layouts():
        wrt_ref[...] = wr_ref[...].T
        for e in range(E):
            qa_flat_ref[:, e * R:(e + 1) * R] = qa_ref[e]
            va_flat_ref[:, e * R:(e + 1) * R] = va_ref[e]

    h = h_ref[...]
    logits = jnp.dot(h, wrt_ref[...], preferred_element_type=jnp.float32)
    m = jnp.max(logits, axis=1, keepdims=True)
    score = 1.0 / jnp.sum(jnp.exp(logits - m), axis=1, keepdims=True)
    idx = jnp.argmax(logits, axis=1)
    col_expert = jax.lax.broadcasted_iota(jnp.int32, (h.shape[0], ER), 1) // R
    mask = jnp.where(col_expert == idx[:, None], score * SCALE, 0.0)
    lr_q = jnp.dot(h, qa_flat_ref[...], preferred_element_type=jnp.float32) * mask
    q_out_ref[...] = jnp.dot(lr_q, qb_ref[...], preferred_element_type=jnp.float32)
    lr_v = jnp.dot(h, va_flat_ref[...], preferred_element_type=jnp.float32) * mask
    v_out_ref[...] = jnp.dot(lr_v, vb_ref[...], preferred_element_type=jnp.float32)


def kernel(hidden_states, router_weight, q_lora_a, q_lora_b, v_lora_a, v_lora_b):
    orig_shape = hidden_states.shape[:-1]
    h = hidden_states.reshape(-1, D)
    T = h.shape[0]
    qb = q_lora_b.reshape(ER, DQ)
    vb = v_lora_b.reshape(ER, DV)

    TS = 512
    grid = (T // TS,)
    q_out, v_out = pl.pallas_call(
        _router_lora_kernel,
        grid=grid,
        in_specs=[
            pl.BlockSpec((TS, D), lambda i: (i, 0)),
            pl.BlockSpec((E, D), lambda i: (0, 0)),
            pl.BlockSpec((E, D, R), lambda i: (0, 0, 0)),
            pl.BlockSpec((ER, DQ), lambda i: (0, 0)),
            pl.BlockSpec((E, D, R), lambda i: (0, 0, 0)),
            pl.BlockSpec((ER, DV), lambda i: (0, 0)),
        ],
        out_specs=[
            pl.BlockSpec((TS, DQ), lambda i: (i, 0)),
            pl.BlockSpec((TS, DV), lambda i: (i, 0)),
        ],
        out_shape=[
            jax.ShapeDtypeStruct((T, DQ), jnp.float32),
            jax.ShapeDtypeStruct((T, DV), jnp.float32),
        ],
        scratch_shapes=[
            pltpu.VMEM((D, E), jnp.float32),
            pltpu.VMEM((D, ER), jnp.float32),
            pltpu.VMEM((D, ER), jnp.float32),
        ],
    )(h, router_weight, q_lora_a, qb, v_lora_a, vb)
    return (q_out.reshape(orig_shape + (DQ,)),
            v_out.reshape(orig_shape + (DV,)))


# speedup vs baseline: 1.2670x; 1.2670x over previous
import jax
import jax.numpy as jnp
from jax.experimental import pallas as pl
from jax.experimental.pallas import tpu as pltpu

E = 8
D = 1024
R = 16
DQ = 1024
DV = 1024
SCALE = 32.0 / 16.0
ER = E * R


def _router_lora_kernel(h_ref, wrt_ref, qa_ref, qb_ref, va_ref, vb_ref,
                        q_out_ref, v_out_ref):
    h = h_ref[...]
    logits = jnp.dot(h, wrt_ref[...], preferred_element_type=jnp.float32)
    m = jnp.max(logits, axis=1, keepdims=True)
    score = 1.0 / jnp.sum(jnp.exp(logits - m), axis=1, keepdims=True)
    idx = jnp.argmax(logits, axis=1)
    col_expert = jax.lax.broadcasted_iota(jnp.int32, (h.shape[0], ER), 1) // R
    mask = jnp.where(col_expert == idx[:, None], score * SCALE, 0.0)
    lr_q = jnp.dot(h, qa_ref[...], preferred_element_type=jnp.float32) * mask
    q_out_ref[...] = jnp.dot(lr_q, qb_ref[...], preferred_element_type=jnp.float32)
    lr_v = jnp.dot(h, va_ref[...], preferred_element_type=jnp.float32) * mask
    v_out_ref[...] = jnp.dot(lr_v, vb_ref[...], preferred_element_type=jnp.float32)


def kernel(hidden_states, router_weight, q_lora_a, q_lora_b, v_lora_a, v_lora_b):
    orig_shape = hidden_states.shape[:-1]
    h = hidden_states.reshape(-1, D)
    T = h.shape[0]
    wrt = router_weight.T
    qa = q_lora_a.transpose(1, 0, 2).reshape(D, ER)
    qb = q_lora_b.reshape(ER, DQ)
    va = v_lora_a.transpose(1, 0, 2).reshape(D, ER)
    vb = v_lora_b.reshape(ER, DV)

    TS = 1024
    grid = (T // TS,)
    q_out, v_out = pl.pallas_call(
        _router_lora_kernel,
        grid=grid,
        in_specs=[
            pl.BlockSpec((TS, D), lambda i: (i, 0)),
            pl.BlockSpec((D, E), lambda i: (0, 0)),
            pl.BlockSpec((D, ER), lambda i: (0, 0)),
            pl.BlockSpec((ER, DQ), lambda i: (0, 0)),
            pl.BlockSpec((D, ER), lambda i: (0, 0)),
            pl.BlockSpec((ER, DV), lambda i: (0, 0)),
        ],
        out_specs=[
            pl.BlockSpec((TS, DQ), lambda i: (i, 0)),
            pl.BlockSpec((TS, DV), lambda i: (i, 0)),
        ],
        out_shape=[
            jax.ShapeDtypeStruct((T, DQ), jnp.float32),
            jax.ShapeDtypeStruct((T, DV), jnp.float32),
        ],
    )(h, wrt, qa, qb, va, vb)
    return (q_out.reshape(orig_shape + (DQ,)),
            v_out.reshape(orig_shape + (DV,)))
